# Initial kernel scaffold; baseline (speedup 1.0000x reference)
import jax
import jax.numpy as jnp
from jax.experimental import pallas as pl


def kernel(xyz, feat, affine_alpha, affine_beta):
    raise NotImplementedError("write your pallas kernel here")


# speedup vs baseline: 1.6784x; 1.6784x over previous
import functools

import jax
import jax.numpy as jnp
from jax import lax
from jax.experimental import pallas as pl
from jax.experimental.pallas import tpu as pltpu

_B, _N, _S, _K, _C = 4, 8192, 1024, 32, 32
_NR = 8
_NL = _N // _NR
_SR, _SL = 8, 128


def _fps_kernel(x_ref, y_ref, z_ref, idx_ref, cx_ref, cy_ref, cz_ref):
    x = x_ref[...]
    y = y_ref[...]
    z = z_ref[...]
    n_iota = (lax.broadcasted_iota(jnp.int32, (_NR, _NL), 0) * _NL
              + lax.broadcasted_iota(jnp.int32, (_NR, _NL), 1))
    s_iota = (lax.broadcasted_iota(jnp.int32, (_SR, _SL), 0) * _SL
              + lax.broadcasted_iota(jnp.int32, (_SR, _SL), 1))

    def body(i, carry):
        dist, far, acci, accx, accy, accz = carry
        ndist, nfar, nai, nax, nay, naz = [], [], [], [], [], []
        for b in range(_B):
            xb = x[8 * b:8 * b + 8]
            yb = y[8 * b:8 * b + 8]
            zb = z[8 * b:8 * b + 8]
            fb = far[b]
            m = n_iota == fb
            cx = jnp.sum(jnp.where(m, xb, 0.0))
            cy = jnp.sum(jnp.where(m, yb, 0.0))
            cz = jnp.sum(jnp.where(m, zb, 0.0))
            sm = s_iota == i
            nai.append(jnp.where(sm, fb, acci[b]))
            nax.append(jnp.where(sm, cx, accx[b]))
            nay.append(jnp.where(sm, cy, accy[b]))
            naz.append(jnp.where(sm, cz, accz[b]))
            dx = xb - cx
            dy = yb - cy
            dz = zb - cz
            d = dx * dx + dy * dy + dz * dz
            db = jnp.minimum(dist[b], d)
            ndist.append(db)
            mx = jnp.max(db)
            nf = jnp.min(jnp.where(db == mx, n_iota, _N)).astype(jnp.int32)
            nfar.append(nf)
        return (tuple(ndist), tuple(nfar), tuple(nai), tuple(nax),
                tuple(nay), tuple(naz))

    dist0 = tuple(jnp.full((_NR, _NL), 1e10, jnp.float32) for _ in range(_B))
    far0 = tuple(jnp.int32(0) for _ in range(_B))
    zf = tuple(jnp.zeros((_SR, _SL), jnp.float32) for _ in range(_B))
    zi = tuple(jnp.zeros((_SR, _SL), jnp.int32) for _ in range(_B))
    carry = lax.fori_loop(
        0, _S, body,
        (dist0, far0, zi, zf,
         tuple(jnp.zeros((_SR, _SL), jnp.float32) for _ in range(_B)),
         tuple(jnp.zeros((_SR, _SL), jnp.float32) for _ in range(_B))))
    _, _, acci, accx, accy, accz = carry
    for b in range(_B):
        idx_ref[b] = acci[b]
        cx_ref[b] = accx[b]
        cy_ref[b] = accy[b]
        cz_ref[b] = accz[b]


def _run_fps(xyz):
    x = xyz[:, :, 0].reshape(_B * _NR, _NL)
    y = xyz[:, :, 1].reshape(_B * _NR, _NL)
    z = xyz[:, :, 2].reshape(_B * _NR, _NL)
    idx, cx, cy, cz = pl.pallas_call(
        _fps_kernel,
        out_shape=[
            jax.ShapeDtypeStruct((_B, _SR, _SL), jnp.int32),
            jax.ShapeDtypeStruct((_B, _SR, _SL), jnp.float32),
            jax.ShapeDtypeStruct((_B, _SR, _SL), jnp.float32),
            jax.ShapeDtypeStruct((_B, _SR, _SL), jnp.float32),
        ],
    )(x, y, z)
    fps_idx = idx.reshape(_B, _S)
    fps_xyz = jnp.stack(
        [cx.reshape(_B, _S), cy.reshape(_B, _S), cz.reshape(_B, _S)], axis=-1)
    return fps_idx, fps_xyz


def kernel(xyz, feat, affine_alpha, affine_beta):
    B, N, _ = xyz.shape
    fps_idx, fps_xyz = _run_fps(xyz)

    batch = jnp.arange(B).reshape(B, 1)
    fps_feat = feat[batch, fps_idx]
    a2 = jnp.sum(fps_xyz ** 2, axis=-1, keepdims=True)
    b2 = jnp.sum(xyz ** 2, axis=-1)[:, None, :]
    ab = jnp.einsum('bsd,bnd->bsn', fps_xyz, xyz)
    dist = jnp.sqrt(jnp.maximum(a2 + b2 - 2.0 * ab, 0.0))
    _, group_idx = lax.top_k(-dist, _K)
    batch3 = jnp.arange(B).reshape(B, 1, 1)
    grouped_feat = feat[batch3, group_idx]
    mean = jnp.mean(grouped_feat, axis=2, keepdims=True)
    diff = grouped_feat - mean
    std = jnp.std(diff.reshape(B, -1), axis=-1, ddof=1).reshape(B, 1, 1, 1)
    grouped_feat = diff / (std + 1e-05)
    grouped_feat = affine_alpha * grouped_feat + affine_beta
    fps_feat_out = jnp.concatenate(
        [grouped_feat,
         jnp.repeat(fps_feat[:, :, None, :], _K, axis=2)], axis=-1)
    return (fps_xyz, fps_feat_out)


# speedup vs baseline: 8.3178x; 4.9557x over previous
import functools

import jax
import jax.numpy as jnp
from jax import lax
from jax.experimental import pallas as pl
from jax.experimental.pallas import tpu as pltpu
from jax.experimental.pallas import tpu_sc as plsc

_B, _N, _S, _K, _C = 4, 8192, 1024, 32, 32
_NR = 8
_NL = _N // _NR
_SR, _SL = 8, 128

_NW = 32
_RPW = (_B * _S) // _NW
_NCH = 32
_CHW = _N // _NCH
_CHV = _CHW // 16


def _fps_kernel(x_ref, y_ref, z_ref, idx_ref, cx_ref, cy_ref, cz_ref):
    x = x_ref[...]
    y = y_ref[...]
    z = z_ref[...]
    n_iota = (lax.broadcasted_iota(jnp.int32, (_NR, _NL), 0) * _NL
              + lax.broadcasted_iota(jnp.int32, (_NR, _NL), 1))
    s_iota = (lax.broadcasted_iota(jnp.int32, (_SR, _SL), 0) * _SL
              + lax.broadcasted_iota(jnp.int32, (_SR, _SL), 1))

    def body(i, carry):
        dist, far, acci, accx, accy, accz = carry
        ndist, nfar, nai, nax, nay, naz = [], [], [], [], [], []
        for b in range(_B):
            xb = x[8 * b:8 * b + 8]
            yb = y[8 * b:8 * b + 8]
            zb = z[8 * b:8 * b + 8]
            fb = far[b]
            m = n_iota == fb
            cx = jnp.sum(jnp.where(m, xb, 0.0))
            cy = jnp.sum(jnp.where(m, yb, 0.0))
            cz = jnp.sum(jnp.where(m, zb, 0.0))
            sm = s_iota == i
            nai.append(jnp.where(sm, fb, acci[b]))
            nax.append(jnp.where(sm, cx, accx[b]))
            nay.append(jnp.where(sm, cy, accy[b]))
            naz.append(jnp.where(sm, cz, accz[b]))
            dx = xb - cx
            dy = yb - cy
            dz = zb - cz
            d = dx * dx + dy * dy + dz * dz
            db = jnp.minimum(dist[b], d)
            ndist.append(db)
            mx = jnp.max(db)
            nf = jnp.min(jnp.where(db == mx, n_iota, _N)).astype(jnp.int32)
            nfar.append(nf)
        return (tuple(ndist), tuple(nfar), tuple(nai), tuple(nax),
                tuple(nay), tuple(naz))

    dist0 = tuple(jnp.full((_NR, _NL), 1e10, jnp.float32) for _ in range(_B))
    far0 = tuple(jnp.int32(0) for _ in range(_B))
    zf = tuple(jnp.zeros((_SR, _SL), jnp.float32) for _ in range(_B))
    zi = tuple(jnp.zeros((_SR, _SL), jnp.int32) for _ in range(_B))
    carry = lax.fori_loop(
        0, _S, body,
        (dist0, far0, zi, zf,
         tuple(jnp.zeros((_SR, _SL), jnp.float32) for _ in range(_B)),
         tuple(jnp.zeros((_SR, _SL), jnp.float32) for _ in range(_B))))
    _, _, acci, accx, accy, accz = carry
    for b in range(_B):
        idx_ref[b] = acci[b]
        cx_ref[b] = accx[b]
        cy_ref[b] = accy[b]
        cz_ref[b] = accz[b]


def _run_fps(xyz):
    x = xyz[:, :, 0].reshape(_B * _NR, _NL)
    y = xyz[:, :, 1].reshape(_B * _NR, _NL)
    z = xyz[:, :, 2].reshape(_B * _NR, _NL)
    idx, cx, cy, cz = pl.pallas_call(
        _fps_kernel,
        out_shape=[
            jax.ShapeDtypeStruct((_B, _SR, _SL), jnp.int32),
            jax.ShapeDtypeStruct((_B, _SR, _SL), jnp.float32),
            jax.ShapeDtypeStruct((_B, _SR, _SL), jnp.float32),
            jax.ShapeDtypeStruct((_B, _SR, _SL), jnp.float32),
        ],
    )(x, y, z)
    return (idx.reshape(_B, _S), cx.reshape(_B, _S), cy.reshape(_B, _S),
            cz.reshape(_B, _S))


def _sc_topk_body(x_hbm, y_hbm, z_hbm, cx_hbm, cy_hbm, cz_hbm, fi_hbm,
                  feat_hbm, grp_hbm, fpf_hbm, gidx_hbm,
                  x_v, y_v, z_v, b2_v, cx_v, cy_v, cz_v, fi_v, d_v, cv_v,
                  ci_v, gi_v, fgi_v, grp_v, fpf_v, sem):
    wid = lax.axis_index("s") * 2 + lax.axis_index("c")
    b = wid // 8
    base = wid * _RPW
    pltpu.sync_copy(x_hbm.at[b], x_v)
    pltpu.sync_copy(y_hbm.at[b], y_v)
    pltpu.sync_copy(z_hbm.at[b], z_v)
    pltpu.sync_copy(cx_hbm.at[pl.ds(base, _RPW)], cx_v)
    pltpu.sync_copy(cy_hbm.at[pl.ds(base, _RPW)], cy_v)
    pltpu.sync_copy(cz_hbm.at[pl.ds(base, _RPW)], cz_v)
    pltpu.sync_copy(fi_hbm.at[pl.ds(base, _RPW)], fi_v)

    lane = lax.broadcasted_iota(jnp.int32, (16,), 0)
    inf = jnp.full((16,), jnp.inf, jnp.float32)
    gbase = b * _N

    def _bf(v):
        u = plsc.bitcast(v, jnp.uint32)
        u = u + jnp.uint32(0x7FFF) + ((u >> 16) & jnp.uint32(1))
        return plsc.bitcast(u & jnp.uint32(0xFFFF0000), jnp.float32)

    def prep(c, _):
        for u in range(_CHV):
            off = c * _CHW + u * 16
            xv = x_v[pl.ds(off, 16)]
            yv = y_v[pl.ds(off, 16)]
            zv = z_v[pl.ds(off, 16)]
            b2_v[pl.ds(off, 16)] = (xv * xv + yv * yv) + zv * zv
            x_v[pl.ds(off, 16)] = _bf(xv)
            y_v[pl.ds(off, 16)] = _bf(yv)
            z_v[pl.ds(off, 16)] = _bf(zv)
        return 0

    lax.fori_loop(0, _NCH, prep, 0, unroll=False)

    _gdn = lax.GatherDimensionNumbers(
        offset_dims=(), collapsed_slice_dims=(0,), start_index_map=(0,))

    def _splat(vec, sub):
        return lax.gather(vec, sub[:, None], dimension_numbers=_gdn,
                          slice_sizes=(1,),
                          mode=lax.GatherScatterMode.PROMISE_IN_BOUNDS)

    def row_body(r, _):
        vbase = (r >> 4) << 4
        sub = jnp.zeros((16,), jnp.int32) + (r & 15)
        cxs = _splat(cx_v[pl.ds(vbase, 16)], sub)
        cys = _splat(cy_v[pl.ds(vbase, 16)], sub)
        czs = _splat(cz_v[pl.ds(vbase, 16)], sub)
        a2s = (cxs * cxs + cys * cys) + czs * czs
        cxb = _bf(cxs)
        cyb = _bf(cys)
        czb = _bf(czs)

        def ch1(c, th):
            cm = inf
            for u in range(_CHV):
                off = c * _CHW + u * 16
                xv = x_v[pl.ds(off, 16)]
                yv = y_v[pl.ds(off, 16)]
                zv = z_v[pl.ds(off, 16)]
                ab = xv * cxb + yv * cyb + zv * czb
                d = jnp.maximum((a2s + b2_v[pl.ds(off, 16)]) - 2.0 * ab, 0.0)
                d_v[pl.ds(off, 16)] = d
                cm = jnp.minimum(cm, d)
            return jnp.maximum(th, jnp.min(cm))

        th = lax.fori_loop(0, _NCH, ch1, -jnp.inf, unroll=False)

        def ch2(c, ptr):
            for u in range(_CHV):
                off = c * _CHW + u * 16
                d = d_v[pl.ds(off, 16)]
                m = d <= th
                mi = m.astype(jnp.int32)
                pos = ptr + plsc.cumsum(mi) - mi
                plsc.store_scatter(ci_v, [pos], off + lane, mask=m)
                plsc.store_scatter(cv_v, [pos], d, mask=m)
                ptr = ptr + plsc.all_reduce_population_count(m)
            return ptr

        ptr = lax.fori_loop(0, _NCH, ch2, jnp.zeros((16,), jnp.int32),
                            unroll=False)
        ncand = jnp.max(ptr)
        plsc.store_scatter(cv_v, [ncand + lane], inf)

        def ins(j, T):
            k0, i0, k1, i1 = T
            vk = cv_v[pl.ds(j * 16, 16)]
            vi = ci_v[pl.ds(j * 16, 16)]
            vk, vi = plsc.sort_key_val(vk, vi)
            rk = lax.rev(vk, (0,))
            ri = lax.rev(vi, (0,))
            m = k1 <= rk
            lk = jnp.where(m, k1, rk)
            li = jnp.where(m, i1, ri)
            lk, li = plsc.sort_key_val(lk, li)
            rk2 = lax.rev(lk, (0,))
            ri2 = lax.rev(li, (0,))
            m2 = k0 <= rk2
            ak = jnp.where(m2, k0, rk2)
            ai = jnp.where(m2, i0, ri2)
            bk = jnp.where(m2, rk2, k0)
            bi = jnp.where(m2, ri2, i0)
            ak, ai = plsc.sort_key_val(ak, ai)
            bk, bi = plsc.sort_key_val(bk, bi)
            return (ak, ai, bk, bi)

        nv = (ncand + 15) >> 4
        zero = jnp.zeros((16,), jnp.int32)
        _, i0, _, i1 = lax.fori_loop(0, nv, ins, (inf, zero, inf, zero),
                                     unroll=False)

        f0 = r * _K + lane
        plsc.store_scatter(gi_v, [f0 >> 7, f0 & 127], i0 + gbase)
        f1 = f0 + 16
        plsc.store_scatter(gi_v, [f1 >> 7, f1 & 127], i1 + gbase)
        return 0

    lax.fori_loop(0, _RPW, row_body, 0, unroll=False)

    for u in range(_RPW // 16):
        fgi_v[pl.ds(u * 16, 16)] = fi_v[pl.ds(u * 16, 16)] + gbase

    cp = pltpu.async_copy(feat_hbm.at[fgi_v], fpf_v, sem)
    cp.wait()
    pltpu.sync_copy(fpf_v, fpf_hbm.at[pl.ds(base, _RPW)])

    def gch(g, _):
        cp = pltpu.async_copy(feat_hbm.at[gi_v.at[g]], grp_v, sem)
        cp.wait()
        pltpu.sync_copy(
            grp_v, grp_hbm.at[pl.ds(wid * (_RPW * _K) + g * 128, 128)])
        return 0

    lax.fori_loop(0, (_RPW * _K) // 128, gch, 0, unroll=False)
    pltpu.sync_copy(gi_v, gidx_hbm.at[pl.ds(wid * 32, 32)])


def _run_sc_topk(x, y, z, cx, cy, cz, fi, feat_flat):
    mesh = plsc.VectorSubcoreMesh(core_axis_name="c", subcore_axis_name="s",
                                  num_cores=2, num_subcores=16)
    f = pl.kernel(
        _sc_topk_body,
        out_type=[
            jax.ShapeDtypeStruct((_B * _S * _K, _C), jnp.float32),
            jax.ShapeDtypeStruct((_B * _S, _C), jnp.float32),
            jax.ShapeDtypeStruct((_B * _S * _K // 128, 128), jnp.int32),
        ],
        mesh=mesh,
        scratch_types=[
            pltpu.VMEM((_N,), jnp.float32),
            pltpu.VMEM((_N,), jnp.float32),
            pltpu.VMEM((_N,), jnp.float32),
            pltpu.VMEM((_N,), jnp.float32),
            pltpu.VMEM((_RPW,), jnp.float32),
            pltpu.VMEM((_RPW,), jnp.float32),
            pltpu.VMEM((_RPW,), jnp.float32),
            pltpu.VMEM((_RPW,), jnp.int32),
            pltpu.VMEM((_N,), jnp.float32),
            pltpu.VMEM((_N + 16,), jnp.float32),
            pltpu.VMEM((_N + 16,), jnp.int32),
            pltpu.VMEM(((_RPW * _K) // 128, 128), jnp.int32),
            pltpu.VMEM((_RPW,), jnp.int32),
            pltpu.VMEM((128, _C), jnp.float32),
            pltpu.VMEM((_RPW, _C), jnp.float32),
            pltpu.SemaphoreType.DMA,
        ],
        compiler_params=pltpu.CompilerParams(needs_layout_passes=False,
                                             use_tc_tiling_on_sc=False),
    )
    return f(x, y, z, cx, cy, cz, fi, feat_flat)


def _norm_kernel(g_ref, ff_ref, at_ref, bt_ref, out_ref):
    g = g_ref[0]
    ff = ff_ref[0]
    kc_i = lax.broadcasted_iota(jnp.int32, (_K * _C, _C), 0)
    kc_j = lax.broadcasted_iota(jnp.int32, (_K * _C, _C), 1)
    m1 = jnp.where(kc_i % _C == kc_j, 1.0 / _K, 0.0)
    mean = jnp.dot(g, m1, preferred_element_type=jnp.float32)
    p_i = lax.broadcasted_iota(jnp.int32, (_C, _K * _C), 0)
    p_j = lax.broadcasted_iota(jnp.int32, (_C, _K * _C), 1)
    p = jnp.where(p_j % _C == p_i, 1.0, 0.0)
    diff = g - jnp.dot(mean, p, preferred_element_type=jnp.float32)
    ss = jnp.sum(diff * diff)
    std = jnp.sqrt(ss / (_S * _K * _C - 1))
    norm = diff * (1.0 / (std + 1e-05)) * at_ref[...] + bt_ref[...]
    q_i = lax.broadcasted_iota(jnp.int32, (_K * _C, 2 * _K * _C), 0)
    q_j = lax.broadcasted_iota(jnp.int32, (_K * _C, 2 * _K * _C), 1)
    q1 = jnp.where(q_j == (q_i // _C) * (2 * _C) + q_i % _C, 1.0, 0.0)
    r_i = lax.broadcasted_iota(jnp.int32, (_C, 2 * _K * _C), 0)
    r_j = lax.broadcasted_iota(jnp.int32, (_C, 2 * _K * _C), 1)
    q2 = jnp.where((r_j % (2 * _C) >= _C) & (r_j % (2 * _C) - _C == r_i),
                   1.0, 0.0)
    out_ref[0] = (jnp.dot(norm, q1, preferred_element_type=jnp.float32)
                  + jnp.dot(ff, q2, preferred_element_type=jnp.float32))


def _run_norm(g, ff, at, bt):
    return pl.pallas_call(
        _norm_kernel,
        grid=(_B,),
        in_specs=[
            pl.BlockSpec((1, _S, _K * _C), lambda b: (b, 0, 0)),
            pl.BlockSpec((1, _S, _C), lambda b: (b, 0, 0)),
            pl.BlockSpec((1, _K * _C), lambda b: (0, 0)),
            pl.BlockSpec((1, _K * _C), lambda b: (0, 0)),
        ],
        out_specs=pl.BlockSpec((1, _S, 2 * _K * _C), lambda b: (b, 0, 0)),
        out_shape=jax.ShapeDtypeStruct((_B, _S, 2 * _K * _C), jnp.float32),
    )(g, ff, at, bt)


def kernel(xyz, feat, affine_alpha, affine_beta):
    fps_idx, cx, cy, cz = _run_fps(xyz)
    fps_xyz = jnp.stack([cx, cy, cz], axis=-1)

    grp, fpf, _ = _run_sc_topk(
        xyz[:, :, 0], xyz[:, :, 1], xyz[:, :, 2],
        cx.reshape(-1), cy.reshape(-1), cz.reshape(-1),
        fps_idx.reshape(-1), feat.reshape(_B * _N, _C))

    at = jnp.tile(affine_alpha.reshape(1, _C), (1, _K))
    bt = jnp.tile(affine_beta.reshape(1, _C), (1, _K))
    out = _run_norm(grp.reshape(_B, _S, _K * _C),
                    fpf.reshape(_B, _S, _C), at, bt)
    return (fps_xyz, out.reshape(_B, _S, _K, 2 * _C))
